# baseline (speedup 1.0000x reference)
import jax
import jax.numpy as jnp
from jax import lax
from jax.experimental import pallas as pl
from jax.experimental.pallas import tpu as pltpu
from jax.experimental.pallas import tpu_sc as plsc
from jax._src.pallas.mosaic import lowering as _mosaic_lowering

_tpu_dialect = _mosaic_lowering.tpu
_ir = _mosaic_lowering.ir

if not getattr(_tpu_dialect, "_w2v_flat_gather_patch", False):
    _orig_enqueue_indirect = _tpu_dialect.enqueue_indirect_dma
    _orig_wait_indirect = _tpu_dialect.wait_indirect_dma

    def _strip_flat_hbm_tiling(ref):
        try:
            ty = _ir.MemRefType(ref.type)
        except Exception:
            return ref
        if (
            len(ty.shape) == 2
            and ty.shape[-1] == 1
            and "hbm" in str(ty.memory_space)
        ):
            base = ref
            try:
                while base.owner.name in ("tpu.memref_reshape",
                                          "tpu.reinterpret_cast"):
                    base = base.owner.operands[0]
            except Exception:
                base = ref
            linear = _ir.Attribute.parse("#tpu.tiled<(1,1),[1,1]>", ty.context)
            new_ty = _ir.MemRefType.get(
                ty.shape, ty.element_type, layout=linear,
                memory_space=ty.memory_space,
            )
            return _tpu_dialect.reinterpret_cast(new_ty, base)
        return ref

    def _enqueue_indirect(source, target, offsets, semaphore, **kw):
        return _orig_enqueue_indirect(
            _strip_flat_hbm_tiling(source), target, offsets, semaphore, **kw
        )

    def _wait_indirect(semaphore, src, dst, **kw):
        return _orig_wait_indirect(
            semaphore, _strip_flat_hbm_tiling(src), dst, **kw
        )

    _tpu_dialect.enqueue_indirect_dma = _enqueue_indirect
    _tpu_dialect.wait_indirect_dma = _wait_indirect

    _orig_memref_reshape = _tpu_dialect.memref_reshape

    def _memref_reshape(result_ty, ref, **kw):
        try:
            rty = _ir.MemRefType(result_ty)
            if len(rty.shape) == 2 and rty.shape[-1] == 1:
                linear = _ir.Attribute.parse(
                    "#tpu.tiled<(1,1),[1,1]>", rty.context
                )
                new_ty = _ir.MemRefType.get(
                    rty.shape, rty.element_type, layout=linear,
                    memory_space=rty.memory_space,
                )
                return _tpu_dialect.reinterpret_cast(new_ty, ref, **kw)
        except Exception:
            pass
        return _orig_memref_reshape(result_ty, ref, **kw)

    _tpu_dialect.memref_reshape = _memref_reshape

    _orig_memref_slice = _tpu_dialect.memref_slice

    def _memref_slice(result_ty, ref, *args, **kw):
        try:
            sty = _ir.MemRefType(ref.type)
            rty = _ir.MemRefType(result_ty)
            if (
                "tiled<(1,1)" in str(sty)
                and "tiled<(1,1)" not in str(rty)
                and len(rty.shape) == len(sty.shape)
            ):
                result_ty = _ir.MemRefType.get(
                    rty.shape, rty.element_type, layout=sty.layout,
                    memory_space=rty.memory_space,
                )
        except Exception:
            pass
        return _orig_memref_slice(result_ty, ref, *args, **kw)

    _tpu_dialect.memref_slice = _memref_slice

    _memref_dialect = _mosaic_lowering.memref
    _orig_alloca = _memref_dialect.alloca

    def _alloca(ty, *args, **kw):
        try:
            mty = _ir.MemRefType(ty)
            if (
                len(mty.shape) == 2
                and mty.shape[-1] == 1
                and "vmem" in str(mty.memory_space)
            ):
                with _ir.Location.unknown(mty.context):
                    linear = _ir.Attribute.parse(
                        "#tpu.tiled<(1,1),[1,1]>", mty.context
                    )
                    ty = _ir.MemRefType.get(
                        mty.shape, mty.element_type, layout=linear,
                        memory_space=mty.memory_space,
                    )
        except Exception:
            pass
        return _orig_alloca(ty, *args, **kw)

    _memref_dialect.alloca = _alloca
    _tpu_dialect._w2v_flat_gather_patch = True

VOC = 1000000
EMBED = 32
BATCH = 16384

_info = plsc.get_sparse_core_info()
_NC, _NS, _L = _info.num_cores, _info.num_subcores, _info.num_lanes
_NW = _NC * _NS
_BPW = BATCH // _NW
_NB = _BPW // _L

_TPB = (VOC + 127) // 128
_BLOCK_WORDS = _TPB * 1024
_FLAT = EMBED * VOC
_COMBO = [a * _BLOCK_WORDS + s * 128 for a in range(EMBED // 8) for s in range(8)]


_CH = 4096
_NCH = EMBED * _BPW // _CH
_RING = 3
_CROWS = _CH // 128


def _sc_body(xt_hbm, xc_hbm, wt_hbm, ct_hbm, out_hbm,
             xt_v, xc_v, idxw, idxc, *rest):
    wbufs = rest[:_RING]
    cbufs = rest[_RING:2 * _RING]
    acc_v = rest[2 * _RING]
    sws = rest[2 * _RING + 1:3 * _RING + 1]
    scs = rest[3 * _RING + 1:4 * _RING + 1]

    wid = lax.axis_index("s") * _NC + lax.axis_index("c")
    base = wid * _BPW

    pltpu.sync_copy(xt_hbm.at[pl.ds(base, _BPW)], xt_v)
    pltpu.sync_copy(xc_hbm.at[pl.ds(base, _BPW)], xc_v)

    def gen(b, carry):
        vt = xt_v[pl.ds(b * _L, _L)]
        vc = xc_v[pl.ds(b * _L, _L)]
        bt = (vt >> 7) * 1024 + (vt & 127)
        bc = (vc >> 7) * 1024 + (vc & 127)
        for k in range(EMBED):
            off = _COMBO[k]
            idxw[pl.ds(k * _BPW + b * _L, _L)] = bt + off
            idxc[pl.ds(k * _BPW + b * _L, _L)] = bc + off
        return carry

    lax.fori_loop(0, _NB, gen, 0, unroll=False)

    def zacc(b, carry):
        acc_v[pl.ds(b * _L, _L)] = jnp.zeros((_L,), jnp.float32)
        return carry

    lax.fori_loop(0, _NB, zacc, 0, unroll=False)

    fw = wt_hbm.reshape(_FLAT, 1)
    fc = ct_hbm.reshape(_FLAT, 1)

    def fire(c, r):
        pltpu.async_copy(fw.at[idxw.at[pl.ds(c * _CH, _CH)]],
                         wbufs[r].reshape(_CH, 1), sws[r])
        pltpu.async_copy(fc.at[idxc.at[pl.ds(c * _CH, _CH)]],
                         cbufs[r].reshape(_CH, 1), scs[r])

    for r in range(_RING):
        fire(r, r)

    def chunk_body(c, r):
        pltpu.make_async_copy(
            fw.at[idxw.at[pl.ds(c * _CH, _CH)]],
            wbufs[r].reshape(_CH, 1), sws[r]).wait()
        pltpu.make_async_copy(
            fc.at[idxc.at[pl.ds(c * _CH, _CH)]],
            cbufs[r].reshape(_CH, 1), scs[r]).wait()

        @pl.when(c + _RING < _NCH)
        def _():
            fire(c + _RING, r)

        def grp(g, carry):
            e = g * _L
            row = e >> 7
            col = e & 127
            gw = wbufs[r][row, pl.ds(col, _L)]
            gc = cbufs[r][row, pl.ds(col, _L)]
            q = e & (_BPW - 1)
            acc_v[pl.ds(q, _L)] = acc_v[pl.ds(q, _L)] + gw * gc
            return carry

        lax.fori_loop(0, _CH // _L, grp, 0, unroll=False)

    def group(g, carry):
        for r in range(_RING):
            chunk_body(g * _RING + r, r)
        return carry

    n_groups = _NCH // _RING
    lax.fori_loop(0, n_groups, group, 0, unroll=False)
    for c in range(n_groups * _RING, _NCH):
        chunk_body(c, c % _RING)

    def sig(b, carry):
        v = acc_v[pl.ds(b * _L, _L)]
        acc_v[pl.ds(b * _L, _L)] = 1.0 / (1.0 + jnp.exp(-v))
        return carry

    lax.fori_loop(0, _NB, sig, 0, unroll=False)

    pltpu.sync_copy(acc_v, out_hbm.at[pl.ds(base, _BPW)])


def kernel(Xt, Xc, W, C):
    mesh = plsc.VectorSubcoreMesh(core_axis_name="c", subcore_axis_name="s")
    f = pl.kernel(
        _sc_body,
        out_type=jax.ShapeDtypeStruct((BATCH,), jnp.float32),
        mesh=mesh,
        compiler_params=pltpu.CompilerParams(
            needs_layout_passes=False, disable_bounds_checks=True),
        scratch_types=(
            [
                pltpu.VMEM((_BPW,), jnp.int32),
                pltpu.VMEM((_BPW,), jnp.int32),
                pltpu.VMEM((EMBED * _BPW,), jnp.int32),
                pltpu.VMEM((EMBED * _BPW,), jnp.int32),
            ]
            + [pltpu.VMEM((_CROWS, 128), jnp.float32) for _ in range(2 * _RING)]
            + [pltpu.VMEM((_BPW,), jnp.float32)]
            + [pltpu.SemaphoreType.DMA for _ in range(2 * _RING)]
        ),
    )
    return f(Xt.astype(jnp.int32), Xc.astype(jnp.int32), W.T, C.T)


# speedup vs baseline: 1.0137x; 1.0137x over previous
import jax
import jax.numpy as jnp
from jax import lax
from jax.experimental import pallas as pl
from jax.experimental.pallas import tpu as pltpu
from jax.experimental.pallas import tpu_sc as plsc
from jax._src.pallas.mosaic import lowering as _mosaic_lowering

_tpu_dialect = _mosaic_lowering.tpu
_ir = _mosaic_lowering.ir

if not getattr(_tpu_dialect, "_w2v_flat_gather_patch", False):
    _orig_enqueue_indirect = _tpu_dialect.enqueue_indirect_dma
    _orig_wait_indirect = _tpu_dialect.wait_indirect_dma

    def _strip_flat_hbm_tiling(ref):
        try:
            ty = _ir.MemRefType(ref.type)
        except Exception:
            return ref
        if (
            len(ty.shape) == 2
            and ty.shape[-1] == 1
            and "hbm" in str(ty.memory_space)
        ):
            base = ref
            try:
                while base.owner.name in ("tpu.memref_reshape",
                                          "tpu.reinterpret_cast"):
                    base = base.owner.operands[0]
            except Exception:
                base = ref
            linear = _ir.Attribute.parse("#tpu.tiled<(1,1),[1,1]>", ty.context)
            new_ty = _ir.MemRefType.get(
                ty.shape, ty.element_type, layout=linear,
                memory_space=ty.memory_space,
            )
            return _tpu_dialect.reinterpret_cast(new_ty, base)
        return ref

    def _enqueue_indirect(source, target, offsets, semaphore, **kw):
        return _orig_enqueue_indirect(
            _strip_flat_hbm_tiling(source), target, offsets, semaphore, **kw
        )

    def _wait_indirect(semaphore, src, dst, **kw):
        return _orig_wait_indirect(
            semaphore, _strip_flat_hbm_tiling(src), dst, **kw
        )

    _tpu_dialect.enqueue_indirect_dma = _enqueue_indirect
    _tpu_dialect.wait_indirect_dma = _wait_indirect

    _orig_memref_reshape = _tpu_dialect.memref_reshape

    def _memref_reshape(result_ty, ref, **kw):
        try:
            rty = _ir.MemRefType(result_ty)
            if len(rty.shape) == 2 and rty.shape[-1] == 1:
                linear = _ir.Attribute.parse(
                    "#tpu.tiled<(1,1),[1,1]>", rty.context
                )
                new_ty = _ir.MemRefType.get(
                    rty.shape, rty.element_type, layout=linear,
                    memory_space=rty.memory_space,
                )
                return _tpu_dialect.reinterpret_cast(new_ty, ref, **kw)
        except Exception:
            pass
        return _orig_memref_reshape(result_ty, ref, **kw)

    _tpu_dialect.memref_reshape = _memref_reshape

    _orig_memref_slice = _tpu_dialect.memref_slice

    def _memref_slice(result_ty, ref, *args, **kw):
        try:
            sty = _ir.MemRefType(ref.type)
            rty = _ir.MemRefType(result_ty)
            if (
                "tiled<(1,1)" in str(sty)
                and "tiled<(1,1)" not in str(rty)
                and len(rty.shape) == len(sty.shape)
            ):
                result_ty = _ir.MemRefType.get(
                    rty.shape, rty.element_type, layout=sty.layout,
                    memory_space=rty.memory_space,
                )
        except Exception:
            pass
        return _orig_memref_slice(result_ty, ref, *args, **kw)

    _tpu_dialect.memref_slice = _memref_slice

    _memref_dialect = _mosaic_lowering.memref
    _orig_alloca = _memref_dialect.alloca

    def _alloca(ty, *args, **kw):
        try:
            mty = _ir.MemRefType(ty)
            if (
                len(mty.shape) == 2
                and mty.shape[-1] == 1
                and "vmem" in str(mty.memory_space)
            ):
                with _ir.Location.unknown(mty.context):
                    linear = _ir.Attribute.parse(
                        "#tpu.tiled<(1,1),[1,1]>", mty.context
                    )
                    ty = _ir.MemRefType.get(
                        mty.shape, mty.element_type, layout=linear,
                        memory_space=mty.memory_space,
                    )
        except Exception:
            pass
        return _orig_alloca(ty, *args, **kw)

    _memref_dialect.alloca = _alloca
    _tpu_dialect._w2v_flat_gather_patch = True

VOC = 1000000
EMBED = 32
BATCH = 16384

_info = plsc.get_sparse_core_info()
_NC, _NS, _L = _info.num_cores, _info.num_subcores, _info.num_lanes
_NW = _NC * _NS
_BPW = BATCH // _NW
_NB = _BPW // _L

_TPB = (VOC + 127) // 128
_BLOCK_WORDS = _TPB * 1024
_FLAT = EMBED * VOC
_COMBO = [a * _BLOCK_WORDS + s * 128 for a in range(EMBED // 8) for s in range(8)]


_CH = 2048
_NCH = EMBED * _BPW // _CH
_RING = 4
_CROWS = _CH // 128


def _sc_body(xt_hbm, xc_hbm, wt_hbm, ct_hbm, out_hbm,
             xt_v, xc_v, idxw, idxc, *rest):
    wbufs = rest[:_RING]
    cbufs = rest[_RING:2 * _RING]
    acc_v = rest[2 * _RING]
    sws = rest[2 * _RING + 1:3 * _RING + 1]
    scs = rest[3 * _RING + 1:4 * _RING + 1]

    wid = lax.axis_index("s") * _NC + lax.axis_index("c")
    base = wid * _BPW

    pltpu.sync_copy(xt_hbm.at[pl.ds(base, _BPW)], xt_v)
    pltpu.sync_copy(xc_hbm.at[pl.ds(base, _BPW)], xc_v)

    def gen(b, carry):
        vt = xt_v[pl.ds(b * _L, _L)]
        vc = xc_v[pl.ds(b * _L, _L)]
        bt = (vt >> 7) * 1024 + (vt & 127)
        bc = (vc >> 7) * 1024 + (vc & 127)
        for k in range(EMBED):
            off = _COMBO[k]
            idxw[pl.ds(k * _BPW + b * _L, _L)] = bt + off
            idxc[pl.ds(k * _BPW + b * _L, _L)] = bc + off
        return carry

    lax.fori_loop(0, _NB, gen, 0, unroll=False)

    def zacc(b, carry):
        acc_v[pl.ds(b * _L, _L)] = jnp.zeros((_L,), jnp.float32)
        return carry

    lax.fori_loop(0, _NB, zacc, 0, unroll=False)

    fw = wt_hbm.reshape(_FLAT, 1)
    fc = ct_hbm.reshape(_FLAT, 1)

    def fire(c, r):
        pltpu.async_copy(fw.at[idxw.at[pl.ds(c * _CH, _CH)]],
                         wbufs[r].reshape(_CH, 1), sws[r])
        pltpu.async_copy(fc.at[idxc.at[pl.ds(c * _CH, _CH)]],
                         cbufs[r].reshape(_CH, 1), scs[r])

    for r in range(_RING):
        fire(r, r)

    def chunk_body(c, r):
        pltpu.make_async_copy(
            fw.at[idxw.at[pl.ds(c * _CH, _CH)]],
            wbufs[r].reshape(_CH, 1), sws[r]).wait()
        pltpu.make_async_copy(
            fc.at[idxc.at[pl.ds(c * _CH, _CH)]],
            cbufs[r].reshape(_CH, 1), scs[r]).wait()

        @pl.when(c + _RING < _NCH)
        def _():
            fire(c + _RING, r)

        def grp(g, carry):
            e = g * _L
            row = e >> 7
            col = e & 127
            gw = wbufs[r][row, pl.ds(col, _L)]
            gc = cbufs[r][row, pl.ds(col, _L)]
            q = e & (_BPW - 1)
            acc_v[pl.ds(q, _L)] = acc_v[pl.ds(q, _L)] + gw * gc
            return carry

        lax.fori_loop(0, _CH // _L, grp, 0, unroll=False)

    def group(g, carry):
        for r in range(_RING):
            chunk_body(g * _RING + r, r)
        return carry

    n_groups = _NCH // _RING
    lax.fori_loop(0, n_groups, group, 0, unroll=False)
    for c in range(n_groups * _RING, _NCH):
        chunk_body(c, c % _RING)

    def sig(b, carry):
        v = acc_v[pl.ds(b * _L, _L)]
        acc_v[pl.ds(b * _L, _L)] = 1.0 / (1.0 + jnp.exp(-v))
        return carry

    lax.fori_loop(0, _NB, sig, 0, unroll=False)

    pltpu.sync_copy(acc_v, out_hbm.at[pl.ds(base, _BPW)])


def kernel(Xt, Xc, W, C):
    mesh = plsc.VectorSubcoreMesh(core_axis_name="c", subcore_axis_name="s")
    f = pl.kernel(
        _sc_body,
        out_type=jax.ShapeDtypeStruct((BATCH,), jnp.float32),
        mesh=mesh,
        compiler_params=pltpu.CompilerParams(
            needs_layout_passes=False, disable_bounds_checks=True),
        scratch_types=(
            [
                pltpu.VMEM((_BPW,), jnp.int32),
                pltpu.VMEM((_BPW,), jnp.int32),
                pltpu.VMEM((EMBED * _BPW,), jnp.int32),
                pltpu.VMEM((EMBED * _BPW,), jnp.int32),
            ]
            + [pltpu.VMEM((_CROWS, 128), jnp.float32) for _ in range(2 * _RING)]
            + [pltpu.VMEM((_BPW,), jnp.float32)]
            + [pltpu.SemaphoreType.DMA for _ in range(2 * _RING)]
        ),
    )
    return f(Xt.astype(jnp.int32), Xc.astype(jnp.int32), W.T, C.T)


# speedup vs baseline: 1.0427x; 1.0286x over previous
import jax
import jax.numpy as jnp
from jax import lax
from jax.experimental import pallas as pl
from jax.experimental.pallas import tpu as pltpu
from jax.experimental.pallas import tpu_sc as plsc
from jax._src.pallas.mosaic import lowering as _mosaic_lowering

_tpu_dialect = _mosaic_lowering.tpu
_ir = _mosaic_lowering.ir

if not getattr(_tpu_dialect, "_w2v_flat_gather_patch", False):
    _orig_enqueue_indirect = _tpu_dialect.enqueue_indirect_dma
    _orig_wait_indirect = _tpu_dialect.wait_indirect_dma

    def _strip_flat_hbm_tiling(ref):
        try:
            ty = _ir.MemRefType(ref.type)
        except Exception:
            return ref
        if (
            len(ty.shape) == 2
            and ty.shape[-1] == 1
            and "hbm" in str(ty.memory_space)
        ):
            base = ref
            try:
                while base.owner.name in ("tpu.memref_reshape",
                                          "tpu.reinterpret_cast"):
                    base = base.owner.operands[0]
            except Exception:
                base = ref
            linear = _ir.Attribute.parse("#tpu.tiled<(1,1),[1,1]>", ty.context)
            new_ty = _ir.MemRefType.get(
                ty.shape, ty.element_type, layout=linear,
                memory_space=ty.memory_space,
            )
            return _tpu_dialect.reinterpret_cast(new_ty, base)
        return ref

    def _enqueue_indirect(source, target, offsets, semaphore, **kw):
        return _orig_enqueue_indirect(
            _strip_flat_hbm_tiling(source), target, offsets, semaphore, **kw
        )

    def _wait_indirect(semaphore, src, dst, **kw):
        return _orig_wait_indirect(
            semaphore, _strip_flat_hbm_tiling(src), dst, **kw
        )

    _tpu_dialect.enqueue_indirect_dma = _enqueue_indirect
    _tpu_dialect.wait_indirect_dma = _wait_indirect

    _orig_memref_reshape = _tpu_dialect.memref_reshape

    def _memref_reshape(result_ty, ref, **kw):
        try:
            rty = _ir.MemRefType(result_ty)
            if len(rty.shape) == 2 and rty.shape[-1] == 1:
                linear = _ir.Attribute.parse(
                    "#tpu.tiled<(1,1),[1,1]>", rty.context
                )
                new_ty = _ir.MemRefType.get(
                    rty.shape, rty.element_type, layout=linear,
                    memory_space=rty.memory_space,
                )
                return _tpu_dialect.reinterpret_cast(new_ty, ref, **kw)
        except Exception:
            pass
        return _orig_memref_reshape(result_ty, ref, **kw)

    _tpu_dialect.memref_reshape = _memref_reshape

    _orig_memref_slice = _tpu_dialect.memref_slice

    def _memref_slice(result_ty, ref, *args, **kw):
        try:
            sty = _ir.MemRefType(ref.type)
            rty = _ir.MemRefType(result_ty)
            if (
                "tiled<(1,1)" in str(sty)
                and "tiled<(1,1)" not in str(rty)
                and len(rty.shape) == len(sty.shape)
            ):
                result_ty = _ir.MemRefType.get(
                    rty.shape, rty.element_type, layout=sty.layout,
                    memory_space=rty.memory_space,
                )
        except Exception:
            pass
        return _orig_memref_slice(result_ty, ref, *args, **kw)

    _tpu_dialect.memref_slice = _memref_slice

    _memref_dialect = _mosaic_lowering.memref
    _orig_alloca = _memref_dialect.alloca

    def _alloca(ty, *args, **kw):
        try:
            mty = _ir.MemRefType(ty)
            if (
                len(mty.shape) == 2
                and mty.shape[-1] == 1
                and "vmem" in str(mty.memory_space)
            ):
                with _ir.Location.unknown(mty.context):
                    linear = _ir.Attribute.parse(
                        "#tpu.tiled<(1,1),[1,1]>", mty.context
                    )
                    ty = _ir.MemRefType.get(
                        mty.shape, mty.element_type, layout=linear,
                        memory_space=mty.memory_space,
                    )
        except Exception:
            pass
        return _orig_alloca(ty, *args, **kw)

    _memref_dialect.alloca = _alloca
    _tpu_dialect._w2v_flat_gather_patch = True

VOC = 1000000
EMBED = 32
BATCH = 16384

_info = plsc.get_sparse_core_info()
_NC, _NS, _L = _info.num_cores, _info.num_subcores, _info.num_lanes
_NW = _NC * _NS
_BPW = BATCH // _NW
_NB = _BPW // _L

_TPB = (VOC + 127) // 128
_BLOCK_WORDS = _TPB * 1024
_FLAT = EMBED * VOC
_COMBO = [a * _BLOCK_WORDS + s * 128 for a in range(EMBED // 8) for s in range(8)]


_CH = 2048
_NCH = EMBED * _BPW // _CH
_RING = 3
_CROWS = _CH // 128


def _sc_body(xt_hbm, xc_hbm, wt_hbm, ct_hbm, out_hbm,
             xt_v, xc_v, idxw, idxc, *rest):
    wbufs = rest[:_RING]
    cbufs = rest[_RING:2 * _RING]
    acc_v = rest[2 * _RING]
    sws = rest[2 * _RING + 1:3 * _RING + 1]
    scs = rest[3 * _RING + 1:4 * _RING + 1]

    wid = lax.axis_index("s") * _NC + lax.axis_index("c")
    base = wid * _BPW

    pltpu.sync_copy(xt_hbm.at[pl.ds(base, _BPW)], xt_v)
    pltpu.sync_copy(xc_hbm.at[pl.ds(base, _BPW)], xc_v)

    def bases(b, carry):
        vt = xt_v[pl.ds(b * _L, _L)]
        vc = xc_v[pl.ds(b * _L, _L)]
        xt_v[pl.ds(b * _L, _L)] = (vt >> 7) * 1024 + (vt & 127)
        xc_v[pl.ds(b * _L, _L)] = (vc >> 7) * 1024 + (vc & 127)
        return carry

    lax.fori_loop(0, _NB, bases, 0, unroll=False)

    def gen_chunk(c):
        def gen(b, carry):
            bt = xt_v[pl.ds(b * _L, _L)]
            bc = xc_v[pl.ds(b * _L, _L)]
            for kk in range(_CH // _BPW):
                k = c * (_CH // _BPW) + kk
                off = _COMBO[k]
                pos = c * _CH + kk * _BPW + b * _L
                idxw[pl.ds(pos, _L)] = bt + off
                idxc[pl.ds(pos, _L)] = bc + off
            return carry

        lax.fori_loop(0, _NB, gen, 0, unroll=False)

    def zacc(b, carry):
        acc_v[pl.ds(b * _L, _L)] = jnp.zeros((_L,), jnp.float32)
        return carry

    lax.fori_loop(0, _NB, zacc, 0, unroll=False)

    fw = wt_hbm.reshape(_FLAT, 1)
    fc = ct_hbm.reshape(_FLAT, 1)

    def fire(c, r):
        pltpu.async_copy(fw.at[idxw.at[pl.ds(c * _CH, _CH)]],
                         wbufs[r].reshape(_CH, 1), sws[r])
        pltpu.async_copy(fc.at[idxc.at[pl.ds(c * _CH, _CH)]],
                         cbufs[r].reshape(_CH, 1), scs[r])

    for r in range(_RING):
        gen_chunk(r)
        fire(r, r)

    def chunk_body(c, r):
        pltpu.make_async_copy(
            fw.at[idxw.at[pl.ds(c * _CH, _CH)]],
            wbufs[r].reshape(_CH, 1), sws[r]).wait()
        pltpu.make_async_copy(
            fc.at[idxc.at[pl.ds(c * _CH, _CH)]],
            cbufs[r].reshape(_CH, 1), scs[r]).wait()

        if c + _RING < _NCH:
            gen_chunk(c + _RING)
            fire(c + _RING, r)

        def grp(g, carry):
            e = g * _L
            row = e >> 7
            col = e & 127
            gw = wbufs[r][row, pl.ds(col, _L)]
            gc = cbufs[r][row, pl.ds(col, _L)]
            q = e & (_BPW - 1)
            acc_v[pl.ds(q, _L)] = acc_v[pl.ds(q, _L)] + gw * gc
            return carry

        lax.fori_loop(0, _CH // _L, grp, 0, unroll=False)

    for c in range(_NCH):
        chunk_body(c, c % _RING)

    def sig(b, carry):
        v = acc_v[pl.ds(b * _L, _L)]
        acc_v[pl.ds(b * _L, _L)] = 1.0 / (1.0 + jnp.exp(-v))
        return carry

    lax.fori_loop(0, _NB, sig, 0, unroll=False)

    pltpu.sync_copy(acc_v, out_hbm.at[pl.ds(base, _BPW)])


def kernel(Xt, Xc, W, C):
    mesh = plsc.VectorSubcoreMesh(core_axis_name="c", subcore_axis_name="s")
    f = pl.kernel(
        _sc_body,
        out_type=jax.ShapeDtypeStruct((BATCH,), jnp.float32),
        mesh=mesh,
        compiler_params=pltpu.CompilerParams(
            needs_layout_passes=False, disable_bounds_checks=True),
        scratch_types=(
            [
                pltpu.VMEM((_BPW,), jnp.int32),
                pltpu.VMEM((_BPW,), jnp.int32),
                pltpu.VMEM((EMBED * _BPW,), jnp.int32),
                pltpu.VMEM((EMBED * _BPW,), jnp.int32),
            ]
            + [pltpu.VMEM((_CROWS, 128), jnp.float32) for _ in range(2 * _RING)]
            + [pltpu.VMEM((_BPW,), jnp.float32)]
            + [pltpu.SemaphoreType.DMA for _ in range(2 * _RING)]
        ),
    )
    return f(Xt.astype(jnp.int32), Xc.astype(jnp.int32), W.T, C.T)
